# baseline (speedup 1.0000x reference)
import functools

import jax
import jax.numpy as jnp
from jax import lax
from jax.experimental import pallas as pl
from jax.experimental.pallas import tpu as pltpu
from jax.experimental.pallas import tpu_sc as plsc

NUM_EMBEDDINGS = 1000000
EMBED_DIM = 64
BATCH = 4096
SEQ = 200

NW = 32
ROWS_PER_W = BATCH // NW
CB = 2
SPLITS = ((0, 128), (128, 72))
CHUNKS = ROWS_PER_W // CB
NB = 2


def _make_kernel():
  mesh = plsc.VectorSubcoreMesh(core_axis_name="c", subcore_axis_name="s")

  @functools.partial(
      pl.kernel,
      mesh=mesh,
      compiler_params=pltpu.CompilerParams(use_tc_tiling_on_sc=False),
      out_type=jax.ShapeDtypeStruct((BATCH, SEQ, EMBED_DIM), jnp.float32),
      scratch_types=[
          pltpu.VMEM((NB, CB, SEQ), jnp.int32),
          pltpu.VMEM((NB, CB, SEQ, EMBED_DIM), jnp.float32),
          pltpu.SemaphoreType.DMA((NB,)),
          pltpu.SemaphoreType.DMA((NB,)),
      ],
  )
  def gather_kernel(idx_hbm, table_hbm, out_hbm, idx_v, rows_v, gsem, wsem):
    wid = lax.axis_index("s") * 2 + lax.axis_index("c")
    base = wid * ROWS_PER_W

    def gathers(b):
      out = []
      for r in range(CB):
        for off, ln in SPLITS:
          out.append((
              table_hbm.at[idx_v.at[b, r, pl.ds(off, ln)]],
              rows_v.at[b, r, pl.ds(off, ln)],
          ))
      return out

    def fire(c, b):
      row0 = base + c * CB
      pltpu.sync_copy(idx_hbm.at[pl.ds(row0, CB)], idx_v.at[b])
      for src, dst in gathers(b):
        pltpu.async_copy(src, dst, gsem.at[b])

    def drain_writeback(c, b):
      row0 = base + c * CB
      for src, dst in gathers(b):
        pltpu.make_async_copy(src, dst, gsem.at[b]).wait()
      pltpu.async_copy(rows_v.at[b], out_hbm.at[pl.ds(row0, CB)], wsem.at[b])

    def wait_writeback(c, b):
      row0 = base + c * CB
      pltpu.make_async_copy(rows_v.at[b], out_hbm.at[pl.ds(row0, CB)],
                            wsem.at[b]).wait()

    fire(0, 0)

    def body(t, _):
      c0 = t * NB
      fire(c0 + 1, 1)
      drain_writeback(c0, 0)
      @pl.when(t + 1 < CHUNKS // NB)
      def _():
        wait_writeback(c0, 0)
        fire(c0 + 2, 0)
      drain_writeback(c0 + 1, 1)
      @pl.when(t + 1 < CHUNKS // NB)
      def _():
        wait_writeback(c0 + 1, 1)
      return 0

    lax.fori_loop(0, CHUNKS // NB, body, 0)
    wait_writeback(CHUNKS - 2, 0)
    wait_writeback(CHUNKS - 1, 1)

  return gather_kernel


_gather = _make_kernel()


@jax.jit
def kernel(indices, table):
  return _gather(indices.astype(jnp.int32), table)


# speedup vs baseline: 1.2425x; 1.2425x over previous
import functools

import jax
import jax.numpy as jnp
from jax import lax
from jax.experimental import pallas as pl
from jax.experimental.pallas import tpu as pltpu
from jax.experimental.pallas import tpu_sc as plsc

NUM_EMBEDDINGS = 1000000
EMBED_DIM = 64
BATCH = 4096
SEQ = 200
PW = 128

NW = 32
ROWS_PER_W = BATCH // NW
CB = 2
SPLITS = ((0, 128), (128, 72))
CHUNKS = ROWS_PER_W // CB
NB = 2


def _make_kernel():
  mesh = plsc.VectorSubcoreMesh(core_axis_name="c", subcore_axis_name="s")

  @functools.partial(
      pl.kernel,
      mesh=mesh,
      compiler_params=pltpu.CompilerParams(use_tc_tiling_on_sc=False),
      out_type=jax.ShapeDtypeStruct((BATCH, SEQ, PW), jnp.float32),
      scratch_types=[
          pltpu.VMEM((NB, CB, SEQ), jnp.int32),
          pltpu.VMEM((NB, CB, SEQ, PW), jnp.float32),
          pltpu.SemaphoreType.DMA((NB,)),
          pltpu.SemaphoreType.DMA((NB,)),
      ],
  )
  def gather_kernel(idx_hbm, table_hbm, out_hbm, idx_v, rows_v, gsem, wsem):
    wid = lax.axis_index("s") * 2 + lax.axis_index("c")
    base = wid * ROWS_PER_W

    def gathers(b):
      out = []
      for r in range(CB):
        for off, ln in SPLITS:
          out.append((
              table_hbm.at[idx_v.at[b, r, pl.ds(off, ln)]],
              rows_v.at[b, r, pl.ds(off, ln)],
          ))
      return out

    def fire(c, b):
      row0 = base + c * CB
      pltpu.sync_copy(idx_hbm.at[pl.ds(row0, CB)], idx_v.at[b])
      for src, dst in gathers(b):
        pltpu.async_copy(src, dst, gsem.at[b])

    def drain_writeback(c, b):
      row0 = base + c * CB
      for src, dst in gathers(b):
        pltpu.make_async_copy(src, dst, gsem.at[b]).wait()
      pltpu.async_copy(rows_v.at[b], out_hbm.at[pl.ds(row0, CB)], wsem.at[b])

    def wait_writeback(c, b):
      row0 = base + c * CB
      pltpu.make_async_copy(rows_v.at[b], out_hbm.at[pl.ds(row0, CB)],
                            wsem.at[b]).wait()

    fire(0, 0)

    def body(t, _):
      c0 = t * NB
      fire(c0 + 1, 1)
      drain_writeback(c0, 0)
      @pl.when(t + 1 < CHUNKS // NB)
      def _():
        wait_writeback(c0, 0)
        fire(c0 + 2, 0)
      drain_writeback(c0 + 1, 1)
      @pl.when(t + 1 < CHUNKS // NB)
      def _():
        wait_writeback(c0 + 1, 1)
      return 0

    lax.fori_loop(0, CHUNKS // NB, body, 0)
    wait_writeback(CHUNKS - 2, 0)
    wait_writeback(CHUNKS - 1, 1)

  return gather_kernel


_gather = _make_kernel()


@jax.jit
def kernel(indices, table):
  table_pad = jnp.pad(table, ((0, 0), (0, PW - EMBED_DIM)))
  out_big = _gather(indices.astype(jnp.int32), table_pad)
  return out_big[:, :, :EMBED_DIM]


# speedup vs baseline: 1.3027x; 1.0484x over previous
import functools

import jax
import jax.numpy as jnp
from jax import lax
from jax.experimental import pallas as pl
from jax.experimental.pallas import tpu as pltpu
from jax.experimental.pallas import tpu_sc as plsc

NUM_EMBEDDINGS = 1000000
EMBED_DIM = 64
BATCH = 4096
SEQ = 200
PW = 128

NW = 32
ROWS_PER_W = BATCH // NW
CB = 2
SPLITS = ((0, 128), (128, 72))
CHUNKS = ROWS_PER_W // CB
NB = 2


def _make_kernel():
  mesh = plsc.VectorSubcoreMesh(core_axis_name="c", subcore_axis_name="s")

  @functools.partial(
      pl.kernel,
      mesh=mesh,
      compiler_params=pltpu.CompilerParams(use_tc_tiling_on_sc=False),
      out_type=jax.ShapeDtypeStruct((BATCH, SEQ, PW), jnp.float32),
      scratch_types=[
          pltpu.VMEM((NB, CB, SEQ), jnp.int32),
          pltpu.VMEM((NB, CB, SEQ, PW), jnp.float32),
          pltpu.SemaphoreType.DMA((NB,)),
          pltpu.SemaphoreType.DMA((NB,)),
      ],
  )
  def gather_kernel(idx_hbm, table_hbm, out_hbm, idx_v, rows_v, gsem, wsem):
    wid = lax.axis_index("s") * 2 + lax.axis_index("c")
    base = wid * ROWS_PER_W

    def gathers(b):
      out = []
      for r in range(CB):
        for off, ln in SPLITS:
          out.append((
              table_hbm.at[idx_v.at[b, r, pl.ds(off, ln)]],
              rows_v.at[b, r, pl.ds(off, ln)],
          ))
      return out

    def fire(c, b):
      row0 = base + c * CB
      pltpu.sync_copy(idx_hbm.at[pl.ds(row0, CB)], idx_v.at[b])
      for src, dst in gathers(b):
        pltpu.async_copy(src, dst, gsem.at[b])

    def drain_writeback(c, b):
      row0 = base + c * CB
      for src, dst in gathers(b):
        pltpu.make_async_copy(src, dst, gsem.at[b]).wait()
      pltpu.async_copy(rows_v.at[b, slice(None), slice(None),
                                 pl.ds(0, EMBED_DIM)],
                       out_hbm.at[pl.ds(row0, CB), slice(None),
                                  pl.ds(0, EMBED_DIM)],
                       wsem.at[b])

    def wait_writeback(c, b):
      row0 = base + c * CB
      pltpu.make_async_copy(rows_v.at[b, slice(None), slice(None),
                                      pl.ds(0, EMBED_DIM)],
                            out_hbm.at[pl.ds(row0, CB), slice(None),
                                       pl.ds(0, EMBED_DIM)],
                            wsem.at[b]).wait()

    fire(0, 0)

    def body(t, _):
      c0 = t * NB
      fire(c0 + 1, 1)
      drain_writeback(c0, 0)
      @pl.when(t + 1 < CHUNKS // NB)
      def _():
        wait_writeback(c0, 0)
        fire(c0 + 2, 0)
      drain_writeback(c0 + 1, 1)
      @pl.when(t + 1 < CHUNKS // NB)
      def _():
        wait_writeback(c0 + 1, 1)
      return 0

    lax.fori_loop(0, CHUNKS // NB, body, 0)
    wait_writeback(CHUNKS - 2, 0)
    wait_writeback(CHUNKS - 1, 1)

  return gather_kernel


_gather = _make_kernel()


@jax.jit
def kernel(indices, table):
  table_pad = jnp.pad(table, ((0, 0), (0, PW - EMBED_DIM)))
  out_big = _gather(indices.astype(jnp.int32), table_pad)
  return out_big[:, :, :EMBED_DIM]
